# Initial kernel scaffold; baseline (speedup 1.0000x reference)
import jax
import jax.numpy as jnp
from jax.experimental import pallas as pl


def kernel(x, edge_index, edge_attr, Wl0, bl0, Wr0, br0, We0, att0, b0, g0, beta0, Wl1, bl1, Wr1, br1, We1, att1, b1, g1, beta1):
    raise NotImplementedError("write your pallas kernel here")


# speedup vs baseline: 1.9328x; 1.9328x over previous
import functools

import jax
import jax.numpy as jnp
from jax import lax
from jax.experimental import pallas as pl
from jax.experimental.pallas import tpu as pltpu
from jax.experimental.pallas import tpu_sc as plsc

N = 10000
E = 320000
IN = 128
H = 4
C = 32
HD = H * C
ED = 16
AD = 2 * HD

NC = 2
NS = 16
HALF = 5120
AROWS = 5376
AR2 = 2 * AROWS
TRASH = HALF
K = 40
ESUB = E // NS
NCHUNK = ESUB // K
RSUB = AR2 // NS


def _proj_pair(h, Wl, bl, Wr, br, br_rows=1000):
    M, D = h.shape
    grid = M // br_rows

    def body(h_ref, wl_ref, bl_ref, wr_ref, brr_ref, xla_ref, xr_ref):
        hh = h_ref[...]
        xla_ref[...] = jnp.dot(hh, wl_ref[...], preferred_element_type=jnp.float32) + bl_ref[...]
        xr_ref[...] = jnp.dot(hh, wr_ref[...], preferred_element_type=jnp.float32) + brr_ref[...]

    return pl.pallas_call(
        body,
        grid=(grid,),
        in_specs=[
            pl.BlockSpec((br_rows, D), lambda i: (i, 0)),
            pl.BlockSpec((D, HD), lambda i: (0, 0)),
            pl.BlockSpec((1, HD), lambda i: (0, 0)),
            pl.BlockSpec((D, HD), lambda i: (0, 0)),
            pl.BlockSpec((1, HD), lambda i: (0, 0)),
        ],
        out_specs=[
            pl.BlockSpec((br_rows, HD), lambda i: (i, 0)),
            pl.BlockSpec((br_rows, HD), lambda i: (i, 0)),
        ],
        out_shape=[
            jax.ShapeDtypeStruct((M, HD), jnp.float32),
            jax.ShapeDtypeStruct((M, HD), jnp.float32),
        ],
    )(h, Wl, bl.reshape(1, HD), Wr, br.reshape(1, HD))


def _edge_proj(ea, We0, We1, dst, br_rows=4000):
    grid = E // br_rows
    rows3 = E // br_rows

    def body(ea_ref, dst_ref, w0_ref, w1_ref, e0_ref, e1_ref, d0a_ref, d0b_ref, d1a_ref, d1b_ref):
        a = ea_ref[...]
        e0_ref[...] = jnp.dot(a, w0_ref[...], preferred_element_type=jnp.float32)
        e1_ref[...] = jnp.dot(a, w1_ref[...], preferred_element_type=jnp.float32)
        d = dst_ref[...]
        l0 = jnp.where(d < HALF, d, TRASH)
        l1r = d - HALF
        l1 = jnp.where(l1r >= 0, l1r, TRASH)
        d0a_ref[...] = 2 * l0
        d0b_ref[...] = 2 * l0 + 1
        d1a_ref[...] = 2 * l1
        d1b_ref[...] = 2 * l1 + 1

    ee0, ee1, d0a, d0b, d1a, d1b = pl.pallas_call(
        body,
        grid=(grid,),
        in_specs=[
            pl.BlockSpec((br_rows, ED), lambda i: (i, 0)),
            pl.BlockSpec((1, 1, br_rows), lambda i: (i, 0, 0)),
            pl.BlockSpec((ED, HD), lambda i: (0, 0)),
            pl.BlockSpec((ED, HD), lambda i: (0, 0)),
        ],
        out_specs=[
            pl.BlockSpec((br_rows, HD), lambda i: (i, 0)),
            pl.BlockSpec((br_rows, HD), lambda i: (i, 0)),
            pl.BlockSpec((1, 1, br_rows), lambda i: (i, 0, 0)),
            pl.BlockSpec((1, 1, br_rows), lambda i: (i, 0, 0)),
            pl.BlockSpec((1, 1, br_rows), lambda i: (i, 0, 0)),
            pl.BlockSpec((1, 1, br_rows), lambda i: (i, 0, 0)),
        ],
        out_shape=[jax.ShapeDtypeStruct((E, HD), jnp.float32)] * 2
        + [jax.ShapeDtypeStruct((grid, 1, br_rows), jnp.int32)] * 4,
    )(ea, dst.reshape(grid, 1, br_rows), We0, We1)
    dlocA = jnp.concatenate([d0a.reshape(E), d1a.reshape(E)])
    dlocB = jnp.concatenate([d0b.reshape(E), d1b.reshape(E)])
    return ee0, ee1, dlocA, dlocB


def _post(num, den, b, g, beta, proj=None, br_rows=1000):
    grid = N // br_rows

    def body(num_ref, den_ref, b_ref, g_ref, beta_ref, *rest):
        nsum = num_ref[...]
        den8 = den_ref[...]
        hi = lax.broadcasted_iota(jnp.int32, (8, HD), 0)
        fi = lax.broadcasted_iota(jnp.int32, (8, HD), 1) // C
        sel = (hi == fi).astype(jnp.float32)
        denr = jnp.dot(den8, sel, preferred_element_type=jnp.float32)
        out = jnp.where(denr > 0, nsum / jnp.maximum(denr, 1e-30), 0.0) + b_ref[...]
        mu = jnp.mean(out, axis=-1, keepdims=True)
        var = jnp.mean((out - mu) ** 2, axis=-1, keepdims=True)
        out = (out - mu) * lax.rsqrt(var + 1e-5) * g_ref[...] + beta_ref[...]
        out = jnp.where(out > 0, out, jnp.exp(jnp.minimum(out, 0.0)) - 1.0)
        if proj is None:
            rest[0][...] = out
        else:
            wl_ref, bl_ref, wr_ref, brr_ref, xla_ref, xr_ref = rest
            xla_ref[...] = jnp.dot(out, wl_ref[...], preferred_element_type=jnp.float32) + bl_ref[...]
            xr_ref[...] = jnp.dot(out, wr_ref[...], preferred_element_type=jnp.float32) + brr_ref[...]

    in_specs = [
        pl.BlockSpec((br_rows, HD), lambda i: (i, 0)),
        pl.BlockSpec((br_rows, 8), lambda i: (i, 0)),
        pl.BlockSpec((1, HD), lambda i: (0, 0)),
        pl.BlockSpec((1, HD), lambda i: (0, 0)),
        pl.BlockSpec((1, HD), lambda i: (0, 0)),
    ]
    args = [num, den, b.reshape(1, HD), g.reshape(1, HD), beta.reshape(1, HD)]
    if proj is None:
        out_specs = [pl.BlockSpec((br_rows, HD), lambda i: (i, 0))]
        out_shape = [jax.ShapeDtypeStruct((N, HD), jnp.float32)]
    else:
        Wl, bl, Wr, br = proj
        in_specs += [
            pl.BlockSpec((HD, HD), lambda i: (0, 0)),
            pl.BlockSpec((1, HD), lambda i: (0, 0)),
            pl.BlockSpec((HD, HD), lambda i: (0, 0)),
            pl.BlockSpec((1, HD), lambda i: (0, 0)),
        ]
        args += [Wl, bl.reshape(1, HD), Wr, br.reshape(1, HD)]
        out_specs = [
            pl.BlockSpec((br_rows, HD), lambda i: (i, 0)),
            pl.BlockSpec((br_rows, HD), lambda i: (i, 0)),
        ]
        out_shape = [jax.ShapeDtypeStruct((N, HD), jnp.float32)] * 2

    return pl.pallas_call(
        body,
        grid=(grid,),
        in_specs=in_specs,
        out_specs=out_specs,
        out_shape=out_shape,
    )(*args)


_mesh = plsc.VectorSubcoreMesh(core_axis_name="c", subcore_axis_name="s")


@functools.partial(
    pl.kernel,
    mesh=_mesh,
    out_type=[jax.ShapeDtypeStruct((NC, AR2, HD), jnp.float32)],
    scratch_types=[
        pltpu.VMEM((K,), jnp.int32),
        pltpu.VMEM((K,), jnp.int32),
        pltpu.VMEM((NC * NS, K), jnp.int32),
        pltpu.VMEM((NC * NS, K), jnp.int32),
        pltpu.VMEM((K, HD), jnp.float32),
        pltpu.VMEM((K, HD), jnp.float32),
        pltpu.VMEM((K, HD), jnp.float32),
        pltpu.VMEM((HD,), jnp.float32),
        pltpu.VMEM_SHARED((AR2, HD), jnp.float32),
        pltpu.SemaphoreType.DMA,
        pltpu.SemaphoreType.DMA,
        pltpu.SemaphoreType.DMA,
    ],
)
def _sc_edge_pass(src_hbm, dst_hbm, dlocA_hbm, dlocB_hbm, xla_hbm, xr_hbm,
                  ee_hbm, att_hbm, zer_hbm, acc_out,
                  srcv, dstv, dst2vA, dst2vB, xjv, xiv, env, attv, acc_sh,
                  sem1, sem2, sem3):
    cid = lax.axis_index("c")
    sid = lax.axis_index("s")
    wid = sid * NC + cid
    rbase = sid * RSUB

    pltpu.sync_copy(zer_hbm.at[pl.ds(0, RSUB)], acc_sh.at[pl.ds(rbase, RSUB)])
    pltpu.sync_copy(zer_hbm.at[pl.ds(0, K)], env)
    pltpu.sync_copy(att_hbm, attv)
    plsc.subcore_barrier()

    att_regs = [attv[pl.ds(j * 16, 16)] for j in range(8)]
    lane = lax.broadcasted_iota(jnp.int32, (16,), 0)
    perms = [lane ^ (1 << p) for p in range(4)]
    ebase = sid * ESUB

    gdn = lax.GatherDimensionNumbers(
        offset_dims=(), collapsed_slice_dims=(0,), start_index_map=(0,))

    def _allsum(v):
        for p in perms:
            v = v + lax.gather(v, p[:, None], dimension_numbers=gdn,
                               slice_sizes=(1,),
                               mode=lax.GatherScatterMode.PROMISE_IN_BOUNDS)
        return v

    def chunk_body(i, carry):
        base = ebase + i * K
        pltpu.sync_copy(src_hbm.at[pl.ds(base, K)], srcv)
        pltpu.sync_copy(dst_hbm.at[pl.ds(base, K)], dstv)
        pltpu.sync_copy(dlocA_hbm.at[pl.ds(cid * E + base, K)], dst2vA.at[wid])
        pltpu.sync_copy(dlocB_hbm.at[pl.ds(cid * E + base, K)], dst2vB.at[wid])
        cp2 = pltpu.async_copy(xr_hbm.at[dstv], xjv, sem2)
        cp3 = pltpu.async_copy(ee_hbm.at[pl.ds(base, K)], xiv, sem3)
        cp2.wait()
        cp3.wait()

        def pre_body(k, carry2):
            for j in range(8):
                xiv[k, pl.ds(j * 16, 16)] = (xiv[k, pl.ds(j * 16, 16)]
                                             + xjv[k, pl.ds(j * 16, 16)])
            return carry2

        lax.fori_loop(0, K, pre_body, 0)
        cp1 = pltpu.async_copy(xla_hbm.at[srcv], xjv, sem1)
        cp1.wait()

        def edge_body(k, carry2):
            xjs = []
            ts = []
            for j in range(8):
                xjj = xjv[k, pl.ds(j * 16, 16)]
                s = xiv[k, pl.ds(j * 16, 16)] + xjj
                s = jnp.maximum(s, s * 0.2)
                xjs.append(xjj)
                ts.append(s * att_regs[j])
            erow = jnp.zeros((16,), jnp.float32)
            for h in range(4):
                a = _allsum(ts[2 * h] + ts[2 * h + 1])
                a = jnp.minimum(jnp.maximum(a, -60.0), 60.0)
                evec = jnp.exp(a)
                xjv[k, pl.ds(2 * h * 16, 16)] = xjs[2 * h] * evec
                xjv[k, pl.ds((2 * h + 1) * 16, 16)] = xjs[2 * h + 1] * evec
                erow = erow + jnp.where(lane == h, evec, 0.0)
            env[k, pl.ds(0, 16)] = erow
            return carry2

        lax.fori_loop(0, K, edge_body, 0)

        pltpu.sync_copy(xjv, acc_sh.at[dst2vA.at[wid]], add=True)
        pltpu.sync_copy(env, acc_sh.at[dst2vB.at[wid]], add=True)
        return carry

    lax.fori_loop(0, NCHUNK, chunk_body, 0)
    plsc.subcore_barrier()

    pltpu.sync_copy(acc_sh.at[pl.ds(rbase, RSUB)], acc_out.at[cid, pl.ds(rbase, RSUB)])


def kernel(x, edge_index, edge_attr, Wl0, bl0, Wr0, br0, We0, att0, b0, g0, beta0,
           Wl1, bl1, Wr1, br1, We1, att1, b1, g1, beta1):
    src = edge_index[0]
    dst = edge_index[1]
    ea = edge_attr.astype(jnp.float32)

    xla0, xr0 = _proj_pair(x, Wl0, bl0, Wr0, br0)
    ee0, ee1, dlocA, dlocB = _edge_proj(ea, We0, We1, dst)

    zer = jnp.zeros((AR2 // NS, HD), jnp.float32)

    def assemble(acc):
        a3 = acc[:, :2 * HALF].reshape(NC, HALF, 2, HD)
        num = jnp.concatenate([a3[0, :, 0], a3[1, :, 0]], axis=0)[:N]
        den = jnp.concatenate([a3[0, :, 1], a3[1, :, 1]], axis=0)[:N, :8]
        return num, den

    (acc0,) = _sc_edge_pass(src, dst, dlocA, dlocB, xla0, xr0, ee0,
                            att0.reshape(HD), zer)
    num0, den0 = assemble(acc0)
    xla1, xr1 = _post(num0, den0, b0, g0, beta0, proj=(Wl1, bl1, Wr1, br1))
    (acc1,) = _sc_edge_pass(src, dst, dlocA, dlocB, xla1, xr1, ee1,
                            att1.reshape(HD), zer)
    num1, den1 = assemble(acc1)
    (h2,) = _post(num1, den1, b1, g1, beta1, proj=None)
    return h2


# speedup vs baseline: 3.0058x; 1.5551x over previous
import functools

import jax
import jax.numpy as jnp
from jax import lax
from jax.experimental import pallas as pl
from jax.experimental.pallas import tpu as pltpu
from jax.experimental.pallas import tpu_sc as plsc

N = 10000
E = 320000
IN = 128
H = 4
C = 32
HD = H * C
ED = 16
AD = 2 * HD

NC = 2
NS = 16
HALF = 5120
AROWS = 5376
AR2 = 2 * AROWS
TRASH = HALF
K = 40
ESUB = E // NS
NCHUNK = ESUB // K
RSUB = AR2 // NS


def _proj_pair(h, Wl, bl, Wr, br, br_rows=1000):
    M, D = h.shape
    grid = M // br_rows

    def body(h_ref, wl_ref, bl_ref, wr_ref, brr_ref, xla_ref, xr_ref):
        hh = h_ref[...]
        xla_ref[...] = jnp.dot(hh, wl_ref[...], preferred_element_type=jnp.float32) + bl_ref[...]
        xr_ref[...] = jnp.dot(hh, wr_ref[...], preferred_element_type=jnp.float32) + brr_ref[...]

    return pl.pallas_call(
        body,
        grid=(grid,),
        in_specs=[
            pl.BlockSpec((br_rows, D), lambda i: (i, 0)),
            pl.BlockSpec((D, HD), lambda i: (0, 0)),
            pl.BlockSpec((1, HD), lambda i: (0, 0)),
            pl.BlockSpec((D, HD), lambda i: (0, 0)),
            pl.BlockSpec((1, HD), lambda i: (0, 0)),
        ],
        out_specs=[
            pl.BlockSpec((br_rows, HD), lambda i: (i, 0)),
            pl.BlockSpec((br_rows, HD), lambda i: (i, 0)),
        ],
        out_shape=[
            jax.ShapeDtypeStruct((M, HD), jnp.float32),
            jax.ShapeDtypeStruct((M, HD), jnp.float32),
        ],
    )(h, Wl, bl.reshape(1, HD), Wr, br.reshape(1, HD))


def _edge_proj(ea, We0, We1, dst, br_rows=4000):
    grid = E // br_rows
    rows3 = E // br_rows

    def body(ea_ref, dst_ref, w0_ref, w1_ref, e0_ref, e1_ref, d0a_ref, d0b_ref, d1a_ref, d1b_ref):
        a = ea_ref[...]
        e0_ref[...] = jnp.dot(a, w0_ref[...], preferred_element_type=jnp.float32)
        e1_ref[...] = jnp.dot(a, w1_ref[...], preferred_element_type=jnp.float32)
        d = dst_ref[...]
        l0 = jnp.where(d < HALF, d, TRASH)
        l1r = d - HALF
        l1 = jnp.where(l1r >= 0, l1r, TRASH)
        d0a_ref[...] = 2 * l0
        d0b_ref[...] = 2 * l0 + 1
        d1a_ref[...] = 2 * l1
        d1b_ref[...] = 2 * l1 + 1

    ee0, ee1, d0a, d0b, d1a, d1b = pl.pallas_call(
        body,
        grid=(grid,),
        in_specs=[
            pl.BlockSpec((br_rows, ED), lambda i: (i, 0)),
            pl.BlockSpec((1, 1, br_rows), lambda i: (i, 0, 0)),
            pl.BlockSpec((ED, HD), lambda i: (0, 0)),
            pl.BlockSpec((ED, HD), lambda i: (0, 0)),
        ],
        out_specs=[
            pl.BlockSpec((br_rows, HD), lambda i: (i, 0)),
            pl.BlockSpec((br_rows, HD), lambda i: (i, 0)),
            pl.BlockSpec((1, 1, br_rows), lambda i: (i, 0, 0)),
            pl.BlockSpec((1, 1, br_rows), lambda i: (i, 0, 0)),
            pl.BlockSpec((1, 1, br_rows), lambda i: (i, 0, 0)),
            pl.BlockSpec((1, 1, br_rows), lambda i: (i, 0, 0)),
        ],
        out_shape=[jax.ShapeDtypeStruct((E, HD), jnp.float32)] * 2
        + [jax.ShapeDtypeStruct((grid, 1, br_rows), jnp.int32)] * 4,
    )(ea, dst.reshape(grid, 1, br_rows), We0, We1)
    dlocA = jnp.concatenate([d0a.reshape(E), d1a.reshape(E)])
    dlocB = jnp.concatenate([d0b.reshape(E), d1b.reshape(E)])
    return ee0, ee1, dlocA, dlocB


def _post(num, den, b, g, beta, proj=None, br_rows=1000):
    grid = N // br_rows

    def body(num_ref, den_ref, b_ref, g_ref, beta_ref, *rest):
        nsum = num_ref[...]
        den8 = den_ref[...]
        hi = lax.broadcasted_iota(jnp.int32, (8, HD), 0)
        fi = lax.broadcasted_iota(jnp.int32, (8, HD), 1) // C
        sel = (hi == fi).astype(jnp.float32)
        denr = jnp.dot(den8, sel, preferred_element_type=jnp.float32)
        out = jnp.where(denr > 0, nsum / jnp.maximum(denr, 1e-30), 0.0) + b_ref[...]
        mu = jnp.mean(out, axis=-1, keepdims=True)
        var = jnp.mean((out - mu) ** 2, axis=-1, keepdims=True)
        out = (out - mu) * lax.rsqrt(var + 1e-5) * g_ref[...] + beta_ref[...]
        out = jnp.where(out > 0, out, jnp.exp(jnp.minimum(out, 0.0)) - 1.0)
        if proj is None:
            rest[0][...] = out
        else:
            wl_ref, bl_ref, wr_ref, brr_ref, xla_ref, xr_ref = rest
            xla_ref[...] = jnp.dot(out, wl_ref[...], preferred_element_type=jnp.float32) + bl_ref[...]
            xr_ref[...] = jnp.dot(out, wr_ref[...], preferred_element_type=jnp.float32) + brr_ref[...]

    in_specs = [
        pl.BlockSpec((br_rows, HD), lambda i: (i, 0)),
        pl.BlockSpec((br_rows, 8), lambda i: (i, 0)),
        pl.BlockSpec((1, HD), lambda i: (0, 0)),
        pl.BlockSpec((1, HD), lambda i: (0, 0)),
        pl.BlockSpec((1, HD), lambda i: (0, 0)),
    ]
    args = [num, den, b.reshape(1, HD), g.reshape(1, HD), beta.reshape(1, HD)]
    if proj is None:
        out_specs = [pl.BlockSpec((br_rows, HD), lambda i: (i, 0))]
        out_shape = [jax.ShapeDtypeStruct((N, HD), jnp.float32)]
    else:
        Wl, bl, Wr, br = proj
        in_specs += [
            pl.BlockSpec((HD, HD), lambda i: (0, 0)),
            pl.BlockSpec((1, HD), lambda i: (0, 0)),
            pl.BlockSpec((HD, HD), lambda i: (0, 0)),
            pl.BlockSpec((1, HD), lambda i: (0, 0)),
        ]
        args += [Wl, bl.reshape(1, HD), Wr, br.reshape(1, HD)]
        out_specs = [
            pl.BlockSpec((br_rows, HD), lambda i: (i, 0)),
            pl.BlockSpec((br_rows, HD), lambda i: (i, 0)),
        ]
        out_shape = [jax.ShapeDtypeStruct((N, HD), jnp.float32)] * 2

    return pl.pallas_call(
        body,
        grid=(grid,),
        in_specs=in_specs,
        out_specs=out_specs,
        out_shape=out_shape,
    )(*args)


_mesh = plsc.VectorSubcoreMesh(core_axis_name="c", subcore_axis_name="s")


@functools.partial(
    pl.kernel,
    mesh=_mesh,
    out_type=[jax.ShapeDtypeStruct((NC, AR2, HD), jnp.float32)],
    scratch_types=[
        pltpu.VMEM((K,), jnp.int32),
        pltpu.VMEM((K,), jnp.int32),
        pltpu.VMEM((NC * NS, K), jnp.int32),
        pltpu.VMEM((NC * NS, K), jnp.int32),
        pltpu.VMEM((K, HD), jnp.float32),
        pltpu.VMEM((K, HD), jnp.float32),
        pltpu.VMEM((K, HD), jnp.float32),
        pltpu.VMEM((K, HD), jnp.float32),
        pltpu.VMEM((HD,), jnp.float32),
        pltpu.VMEM_SHARED((AR2, HD), jnp.float32),
        pltpu.SemaphoreType.DMA,
        pltpu.SemaphoreType.DMA,
        pltpu.SemaphoreType.DMA,
        pltpu.SemaphoreType.DMA,
    ],
)
def _sc_edge_pass(src_hbm, dst_hbm, dlocA_hbm, dlocB_hbm, xla_hbm, xr_hbm,
                  ee_hbm, att_hbm, zer_hbm, acc_out,
                  srcv, dstv, dst2vA, dst2vB, xjv, xiv, eev, env, attv, acc_sh,
                  sem1, sem2, sem3, sem4):
    cid = lax.axis_index("c")
    sid = lax.axis_index("s")
    wid = sid * NC + cid
    rbase = sid * RSUB

    pltpu.sync_copy(zer_hbm.at[pl.ds(0, RSUB)], acc_sh.at[pl.ds(rbase, RSUB)])
    pltpu.sync_copy(zer_hbm.at[pl.ds(0, K)], env)
    pltpu.sync_copy(att_hbm, attv)
    plsc.subcore_barrier()

    att_regs = [attv[pl.ds(j * 16, 16)] for j in range(8)]
    lane = lax.broadcasted_iota(jnp.int32, (16,), 0)
    perms = [lane ^ (1 << p) for p in range(4)]
    ebase = sid * ESUB

    gdn = lax.GatherDimensionNumbers(
        offset_dims=(), collapsed_slice_dims=(0,), start_index_map=(0,))

    def _allsum(v):
        for p in perms:
            v = v + lax.gather(v, p[:, None], dimension_numbers=gdn,
                               slice_sizes=(1,),
                               mode=lax.GatherScatterMode.PROMISE_IN_BOUNDS)
        return v

    def chunk_body(i, carry):
        base = ebase + i * K
        ci1 = pltpu.async_copy(src_hbm.at[pl.ds(base, K)], srcv, sem1)
        ci2 = pltpu.async_copy(dst_hbm.at[pl.ds(base, K)], dstv, sem2)
        ci3 = pltpu.async_copy(dlocA_hbm.at[pl.ds(cid * E + base, K)],
                               dst2vA.at[wid], sem3)
        ci4 = pltpu.async_copy(dlocB_hbm.at[pl.ds(cid * E + base, K)],
                               dst2vB.at[wid], sem4)
        ci3.wait()
        ci4.wait()
        cp3 = pltpu.async_copy(ee_hbm.at[pl.ds(base, K)], eev, sem3)
        ci1.wait()
        cp1 = pltpu.async_copy(xla_hbm.at[srcv], xjv, sem1)
        ci2.wait()
        cp2 = pltpu.async_copy(xr_hbm.at[dstv], xiv, sem2)
        cp1.wait()
        cp2.wait()
        cp3.wait()

        def edge_body(k, carry2):
            xjs = []
            ts = []
            for j in range(8):
                xjj = xjv[k, pl.ds(j * 16, 16)]
                s = xiv[k, pl.ds(j * 16, 16)] + xjj + eev[k, pl.ds(j * 16, 16)]
                s = jnp.maximum(s, s * 0.2)
                xjs.append(xjj)
                ts.append(s * att_regs[j])
            erow = jnp.zeros((16,), jnp.float32)
            for h in range(4):
                a = _allsum(ts[2 * h] + ts[2 * h + 1])
                a = jnp.minimum(jnp.maximum(a, -60.0), 60.0)
                evec = jnp.exp(a)
                xjv[k, pl.ds(2 * h * 16, 16)] = xjs[2 * h] * evec
                xjv[k, pl.ds((2 * h + 1) * 16, 16)] = xjs[2 * h + 1] * evec
                erow = erow + jnp.where(lane == h, evec, 0.0)
            env[k, pl.ds(0, 16)] = erow
            return carry2

        lax.fori_loop(0, K, edge_body, 0)

        pltpu.sync_copy(xjv, acc_sh.at[dst2vA.at[wid]], add=True)
        pltpu.sync_copy(env, acc_sh.at[dst2vB.at[wid]], add=True)
        return carry

    lax.fori_loop(0, NCHUNK, chunk_body, 0)
    plsc.subcore_barrier()

    pltpu.sync_copy(acc_sh.at[pl.ds(rbase, RSUB)], acc_out.at[cid, pl.ds(rbase, RSUB)])


def kernel(x, edge_index, edge_attr, Wl0, bl0, Wr0, br0, We0, att0, b0, g0, beta0,
           Wl1, bl1, Wr1, br1, We1, att1, b1, g1, beta1):
    src = edge_index[0]
    dst = edge_index[1]
    ea = edge_attr.astype(jnp.float32)

    xla0, xr0 = _proj_pair(x, Wl0, bl0, Wr0, br0)
    ee0, ee1, dlocA, dlocB = _edge_proj(ea, We0, We1, dst)

    zer = jnp.zeros((AR2 // NS, HD), jnp.float32)

    def assemble(acc):
        a3 = acc[:, :2 * HALF].reshape(NC, HALF, 2, HD)
        num = jnp.concatenate([a3[0, :, 0], a3[1, :, 0]], axis=0)[:N]
        den = jnp.concatenate([a3[0, :, 1], a3[1, :, 1]], axis=0)[:N, :8]
        return num, den

    (acc0,) = _sc_edge_pass(src, dst, dlocA, dlocB, xla0, xr0, ee0,
                            att0.reshape(HD), zer)
    num0, den0 = assemble(acc0)
    xla1, xr1 = _post(num0, den0, b0, g0, beta0, proj=(Wl1, bl1, Wr1, br1))
    (acc1,) = _sc_edge_pass(src, dst, dlocA, dlocB, xla1, xr1, ee1,
                            att1.reshape(HD), zer)
    num1, den1 = assemble(acc1)
    (h2,) = _post(num1, den1, b1, g1, beta1, proj=None)
    return h2


# speedup vs baseline: 3.0594x; 1.0178x over previous
import functools

import jax
import jax.numpy as jnp
from jax import lax
from jax.experimental import pallas as pl
from jax.experimental.pallas import tpu as pltpu
from jax.experimental.pallas import tpu_sc as plsc

N = 10000
E = 320000
IN = 128
H = 4
C = 32
HD = H * C
ED = 16
AD = 2 * HD

NC = 2
NS = 16
HALF = 5120
AROWS = 5376
AR2 = 2 * AROWS
TRASH = HALF
K = 40
ESUB = E // NS
NCHUNK = ESUB // K
RSUB = AR2 // NS


def _proj_pair(h, Wl, bl, Wr, br, br_rows=1000):
    M, D = h.shape
    grid = M // br_rows

    def body(h_ref, wl_ref, bl_ref, wr_ref, brr_ref, xla_ref, xr_ref):
        hh = h_ref[...]
        xla_ref[...] = jnp.dot(hh, wl_ref[...], preferred_element_type=jnp.float32) + bl_ref[...]
        xr_ref[...] = jnp.dot(hh, wr_ref[...], preferred_element_type=jnp.float32) + brr_ref[...]

    return pl.pallas_call(
        body,
        grid=(grid,),
        in_specs=[
            pl.BlockSpec((br_rows, D), lambda i: (i, 0)),
            pl.BlockSpec((D, HD), lambda i: (0, 0)),
            pl.BlockSpec((1, HD), lambda i: (0, 0)),
            pl.BlockSpec((D, HD), lambda i: (0, 0)),
            pl.BlockSpec((1, HD), lambda i: (0, 0)),
        ],
        out_specs=[
            pl.BlockSpec((br_rows, HD), lambda i: (i, 0)),
            pl.BlockSpec((br_rows, HD), lambda i: (i, 0)),
        ],
        out_shape=[
            jax.ShapeDtypeStruct((M, HD), jnp.float32),
            jax.ShapeDtypeStruct((M, HD), jnp.float32),
        ],
    )(h, Wl, bl.reshape(1, HD), Wr, br.reshape(1, HD))


def _edge_proj(ea, We0, We1, dst, br_rows=4000):
    grid = E // br_rows
    rows3 = E // br_rows

    def body(ea_ref, dst_ref, w0_ref, w1_ref, e0_ref, e1_ref, d0a_ref, d0b_ref, d1a_ref, d1b_ref):
        a = ea_ref[...]
        e0_ref[...] = jnp.dot(a, w0_ref[...], preferred_element_type=jnp.float32)
        e1_ref[...] = jnp.dot(a, w1_ref[...], preferred_element_type=jnp.float32)
        d = dst_ref[...]
        l0 = jnp.where(d < HALF, d, TRASH)
        l1r = d - HALF
        l1 = jnp.where(l1r >= 0, l1r, TRASH)
        d0a_ref[...] = 2 * l0
        d0b_ref[...] = 2 * l0 + 1
        d1a_ref[...] = 2 * l1
        d1b_ref[...] = 2 * l1 + 1

    ee0, ee1, d0a, d0b, d1a, d1b = pl.pallas_call(
        body,
        grid=(grid,),
        in_specs=[
            pl.BlockSpec((br_rows, ED), lambda i: (i, 0)),
            pl.BlockSpec((1, 1, br_rows), lambda i: (i, 0, 0)),
            pl.BlockSpec((ED, HD), lambda i: (0, 0)),
            pl.BlockSpec((ED, HD), lambda i: (0, 0)),
        ],
        out_specs=[
            pl.BlockSpec((br_rows, HD), lambda i: (i, 0)),
            pl.BlockSpec((br_rows, HD), lambda i: (i, 0)),
            pl.BlockSpec((1, 1, br_rows), lambda i: (i, 0, 0)),
            pl.BlockSpec((1, 1, br_rows), lambda i: (i, 0, 0)),
            pl.BlockSpec((1, 1, br_rows), lambda i: (i, 0, 0)),
            pl.BlockSpec((1, 1, br_rows), lambda i: (i, 0, 0)),
        ],
        out_shape=[jax.ShapeDtypeStruct((E, HD), jnp.float32)] * 2
        + [jax.ShapeDtypeStruct((grid, 1, br_rows), jnp.int32)] * 4,
    )(ea, dst.reshape(grid, 1, br_rows), We0, We1)
    dlocA = jnp.concatenate([d0a.reshape(E), d1a.reshape(E)])
    dlocB = jnp.concatenate([d0b.reshape(E), d1b.reshape(E)])
    return ee0, ee1, dlocA, dlocB


def _post(num, den, b, g, beta, proj=None, br_rows=1000):
    grid = N // br_rows

    def body(num_ref, den_ref, b_ref, g_ref, beta_ref, *rest):
        nsum = num_ref[...]
        den8 = den_ref[...]
        hi = lax.broadcasted_iota(jnp.int32, (8, HD), 0)
        fi = lax.broadcasted_iota(jnp.int32, (8, HD), 1) // C
        sel = (hi == fi).astype(jnp.float32)
        denr = jnp.dot(den8, sel, preferred_element_type=jnp.float32)
        out = jnp.where(denr > 0, nsum / jnp.maximum(denr, 1e-30), 0.0) + b_ref[...]
        mu = jnp.mean(out, axis=-1, keepdims=True)
        var = jnp.mean((out - mu) ** 2, axis=-1, keepdims=True)
        out = (out - mu) * lax.rsqrt(var + 1e-5) * g_ref[...] + beta_ref[...]
        out = jnp.where(out > 0, out, jnp.exp(jnp.minimum(out, 0.0)) - 1.0)
        if proj is None:
            rest[0][...] = out
        else:
            wl_ref, bl_ref, wr_ref, brr_ref, xla_ref, xr_ref = rest
            xla_ref[...] = jnp.dot(out, wl_ref[...], preferred_element_type=jnp.float32) + bl_ref[...]
            xr_ref[...] = jnp.dot(out, wr_ref[...], preferred_element_type=jnp.float32) + brr_ref[...]

    in_specs = [
        pl.BlockSpec((br_rows, HD), lambda i: (i, 0)),
        pl.BlockSpec((br_rows, 8), lambda i: (i, 0)),
        pl.BlockSpec((1, HD), lambda i: (0, 0)),
        pl.BlockSpec((1, HD), lambda i: (0, 0)),
        pl.BlockSpec((1, HD), lambda i: (0, 0)),
    ]
    args = [num, den, b.reshape(1, HD), g.reshape(1, HD), beta.reshape(1, HD)]
    if proj is None:
        out_specs = [pl.BlockSpec((br_rows, HD), lambda i: (i, 0))]
        out_shape = [jax.ShapeDtypeStruct((N, HD), jnp.float32)]
    else:
        Wl, bl, Wr, br = proj
        in_specs += [
            pl.BlockSpec((HD, HD), lambda i: (0, 0)),
            pl.BlockSpec((1, HD), lambda i: (0, 0)),
            pl.BlockSpec((HD, HD), lambda i: (0, 0)),
            pl.BlockSpec((1, HD), lambda i: (0, 0)),
        ]
        args += [Wl, bl.reshape(1, HD), Wr, br.reshape(1, HD)]
        out_specs = [
            pl.BlockSpec((br_rows, HD), lambda i: (i, 0)),
            pl.BlockSpec((br_rows, HD), lambda i: (i, 0)),
        ]
        out_shape = [jax.ShapeDtypeStruct((N, HD), jnp.float32)] * 2

    return pl.pallas_call(
        body,
        grid=(grid,),
        in_specs=in_specs,
        out_specs=out_specs,
        out_shape=out_shape,
    )(*args)


_mesh = plsc.VectorSubcoreMesh(core_axis_name="c", subcore_axis_name="s")


@functools.partial(
    pl.kernel,
    mesh=_mesh,
    out_type=[jax.ShapeDtypeStruct((NC, AR2, HD), jnp.float32)],
    scratch_types=[
        pltpu.VMEM((K,), jnp.int32),
        pltpu.VMEM((K,), jnp.int32),
        pltpu.VMEM((NC * NS, K), jnp.int32),
        pltpu.VMEM((NC * NS, K), jnp.int32),
        pltpu.VMEM((K, HD), jnp.float32),
        pltpu.VMEM((K, HD), jnp.float32),
        pltpu.VMEM((K, HD), jnp.float32),
        pltpu.VMEM((K, HD), jnp.float32),
        pltpu.VMEM((HD,), jnp.float32),
        pltpu.VMEM_SHARED((AR2, HD), jnp.float32),
        pltpu.SemaphoreType.DMA,
        pltpu.SemaphoreType.DMA,
        pltpu.SemaphoreType.DMA,
        pltpu.SemaphoreType.DMA,
    ],
)
def _sc_edge_pass(src_hbm, dst_hbm, dlocA_hbm, dlocB_hbm, xla_hbm, xr_hbm,
                  ee_hbm, att_hbm, zer_hbm, acc_out,
                  srcv, dstv, dst2vA, dst2vB, xjv, xiv, eev, env, attv, acc_sh,
                  sem1, sem2, sem3, sem4):
    cid = lax.axis_index("c")
    sid = lax.axis_index("s")
    wid = sid * NC + cid
    rbase = sid * RSUB

    pltpu.sync_copy(zer_hbm.at[pl.ds(0, RSUB)], acc_sh.at[pl.ds(rbase, RSUB)])
    pltpu.sync_copy(zer_hbm.at[pl.ds(0, K)], env)
    pltpu.sync_copy(att_hbm, attv)
    plsc.subcore_barrier()

    att_regs = [attv[pl.ds(j * 16, 16)] for j in range(8)]
    lane = lax.broadcasted_iota(jnp.int32, (16,), 0)
    perms = [lane ^ (1 << p) for p in range(4)]
    ebase = sid * ESUB

    gdn = lax.GatherDimensionNumbers(
        offset_dims=(), collapsed_slice_dims=(0,), start_index_map=(0,))

    def _allsum(v):
        for p in perms:
            v = v + lax.gather(v, p[:, None], dimension_numbers=gdn,
                               slice_sizes=(1,),
                               mode=lax.GatherScatterMode.PROMISE_IN_BOUNDS)
        return v

    def chunk_body(i, carry):
        base = ebase + i * K
        ci1 = pltpu.async_copy(src_hbm.at[pl.ds(base, K)], srcv, sem1)
        ci2 = pltpu.async_copy(dst_hbm.at[pl.ds(base, K)], dstv, sem2)
        ci3 = pltpu.async_copy(dlocA_hbm.at[pl.ds(cid * E + base, K)],
                               dst2vA.at[wid], sem3)
        ci4 = pltpu.async_copy(dlocB_hbm.at[pl.ds(cid * E + base, K)],
                               dst2vB.at[wid], sem4)
        ci3.wait()
        ci4.wait()
        cp3 = pltpu.async_copy(ee_hbm.at[pl.ds(base, K)], eev, sem3)
        ci1.wait()
        cp1 = pltpu.async_copy(xla_hbm.at[srcv], xjv, sem1)
        ci2.wait()
        cp2 = pltpu.async_copy(xr_hbm.at[dstv], xiv, sem2)
        cp1.wait()
        cp2.wait()
        cp3.wait()

        def edge_body(k, carry2):
            xjs = []
            ts = []
            for j in range(8):
                xjj = xjv[k, pl.ds(j * 16, 16)]
                s = xiv[k, pl.ds(j * 16, 16)] + xjj + eev[k, pl.ds(j * 16, 16)]
                s = jnp.maximum(s, s * 0.2)
                xjs.append(xjj)
                ts.append(s * att_regs[j])
            erow = jnp.zeros((16,), jnp.float32)
            for h in range(4):
                a = _allsum(ts[2 * h] + ts[2 * h + 1])
                a = jnp.minimum(jnp.maximum(a, -60.0), 60.0)
                evec = jnp.exp(a)
                xjv[k, pl.ds(2 * h * 16, 16)] = xjs[2 * h] * evec
                xjv[k, pl.ds((2 * h + 1) * 16, 16)] = xjs[2 * h + 1] * evec
                erow = erow + jnp.where(lane == h, evec, 0.0)
            env[k, pl.ds(0, 16)] = erow
            return carry2

        lax.fori_loop(0, K, edge_body, 0)

        cs1 = pltpu.async_copy(xjv, acc_sh.at[dst2vA.at[wid]], sem1, add=True)
        cs2 = pltpu.async_copy(env, acc_sh.at[dst2vB.at[wid]], sem2, add=True)
        cs1.wait()
        cs2.wait()
        return carry

    lax.fori_loop(0, NCHUNK, chunk_body, 0)
    plsc.subcore_barrier()

    pltpu.sync_copy(acc_sh.at[pl.ds(rbase, RSUB)], acc_out.at[cid, pl.ds(rbase, RSUB)])


def kernel(x, edge_index, edge_attr, Wl0, bl0, Wr0, br0, We0, att0, b0, g0, beta0,
           Wl1, bl1, Wr1, br1, We1, att1, b1, g1, beta1):
    src = edge_index[0]
    dst = edge_index[1]
    ea = edge_attr.astype(jnp.float32)

    xla0, xr0 = _proj_pair(x, Wl0, bl0, Wr0, br0)
    ee0, ee1, dlocA, dlocB = _edge_proj(ea, We0, We1, dst)

    zer = jnp.zeros((AR2 // NS, HD), jnp.float32)

    def assemble(acc):
        a3 = acc[:, :2 * HALF].reshape(NC, HALF, 2, HD)
        num = jnp.concatenate([a3[0, :, 0], a3[1, :, 0]], axis=0)[:N]
        den = jnp.concatenate([a3[0, :, 1], a3[1, :, 1]], axis=0)[:N, :8]
        return num, den

    (acc0,) = _sc_edge_pass(src, dst, dlocA, dlocB, xla0, xr0, ee0,
                            att0.reshape(HD), zer)
    num0, den0 = assemble(acc0)
    xla1, xr1 = _post(num0, den0, b0, g0, beta0, proj=(Wl1, bl1, Wr1, br1))
    (acc1,) = _sc_edge_pass(src, dst, dlocA, dlocB, xla1, xr1, ee1,
                            att1.reshape(HD), zer)
    num1, den1 = assemble(acc1)
    (h2,) = _post(num1, den1, b1, g1, beta1, proj=None)
    return h2


# speedup vs baseline: 3.2462x; 1.0611x over previous
import functools

import jax
import jax.numpy as jnp
from jax import lax
from jax.experimental import pallas as pl
from jax.experimental.pallas import tpu as pltpu
from jax.experimental.pallas import tpu_sc as plsc

N = 10000
E = 320000
IN = 128
H = 4
C = 32
HD = H * C
ED = 16
AD = 2 * HD

NC = 2
NS = 16
HALF = 5120
AROWS = 5376
AR2 = 2 * AROWS
TRASH = HALF
K = 40
ESUB = E // NS
NCHUNK = ESUB // K
RSUB = AR2 // NS


def _proj_pair(h, Wl, bl, Wr, br, br_rows=1000):
    M, D = h.shape
    grid = M // br_rows

    def body(h_ref, wl_ref, bl_ref, wr_ref, brr_ref, xla_ref, xr_ref):
        hh = h_ref[...]
        xla_ref[...] = jnp.dot(hh, wl_ref[...], preferred_element_type=jnp.float32) + bl_ref[...]
        xr_ref[...] = jnp.dot(hh, wr_ref[...], preferred_element_type=jnp.float32) + brr_ref[...]

    return pl.pallas_call(
        body,
        grid=(grid,),
        in_specs=[
            pl.BlockSpec((br_rows, D), lambda i: (i, 0)),
            pl.BlockSpec((D, HD), lambda i: (0, 0)),
            pl.BlockSpec((1, HD), lambda i: (0, 0)),
            pl.BlockSpec((D, HD), lambda i: (0, 0)),
            pl.BlockSpec((1, HD), lambda i: (0, 0)),
        ],
        out_specs=[
            pl.BlockSpec((br_rows, HD), lambda i: (i, 0)),
            pl.BlockSpec((br_rows, HD), lambda i: (i, 0)),
        ],
        out_shape=[
            jax.ShapeDtypeStruct((M, HD), jnp.float32),
            jax.ShapeDtypeStruct((M, HD), jnp.float32),
        ],
    )(h, Wl, bl.reshape(1, HD), Wr, br.reshape(1, HD))


def _edge_proj(ea, We0, We1, dst, br_rows=4000):
    grid = E // br_rows
    rows3 = E // br_rows

    def body(ea_ref, dst_ref, w0_ref, w1_ref, e0_ref, e1_ref, d0a_ref, d0b_ref, d1a_ref, d1b_ref):
        a = ea_ref[...]
        e0_ref[...] = jnp.dot(a, w0_ref[...], preferred_element_type=jnp.float32)
        e1_ref[...] = jnp.dot(a, w1_ref[...], preferred_element_type=jnp.float32)
        d = dst_ref[...]
        l0 = jnp.where(d < HALF, d, TRASH)
        l1r = d - HALF
        l1 = jnp.where(l1r >= 0, l1r, TRASH)
        d0a_ref[...] = 2 * l0
        d0b_ref[...] = 2 * l0 + 1
        d1a_ref[...] = 2 * l1
        d1b_ref[...] = 2 * l1 + 1

    ee0, ee1, d0a, d0b, d1a, d1b = pl.pallas_call(
        body,
        grid=(grid,),
        in_specs=[
            pl.BlockSpec((br_rows, ED), lambda i: (i, 0)),
            pl.BlockSpec((1, 1, br_rows), lambda i: (i, 0, 0)),
            pl.BlockSpec((ED, HD), lambda i: (0, 0)),
            pl.BlockSpec((ED, HD), lambda i: (0, 0)),
        ],
        out_specs=[
            pl.BlockSpec((br_rows, HD), lambda i: (i, 0)),
            pl.BlockSpec((br_rows, HD), lambda i: (i, 0)),
            pl.BlockSpec((1, 1, br_rows), lambda i: (i, 0, 0)),
            pl.BlockSpec((1, 1, br_rows), lambda i: (i, 0, 0)),
            pl.BlockSpec((1, 1, br_rows), lambda i: (i, 0, 0)),
            pl.BlockSpec((1, 1, br_rows), lambda i: (i, 0, 0)),
        ],
        out_shape=[jax.ShapeDtypeStruct((E, HD), jnp.float32)] * 2
        + [jax.ShapeDtypeStruct((grid, 1, br_rows), jnp.int32)] * 4,
    )(ea, dst.reshape(grid, 1, br_rows), We0, We1)
    dlocA = jnp.concatenate([d0a.reshape(E), d1a.reshape(E)])
    dlocB = jnp.concatenate([d0b.reshape(E), d1b.reshape(E)])
    return ee0, ee1, dlocA, dlocB


def _post(num, den, b, g, beta, proj=None, br_rows=1000):
    grid = N // br_rows

    def body(num_ref, den_ref, b_ref, g_ref, beta_ref, *rest):
        nsum = num_ref[...]
        den8 = den_ref[...]
        hi = lax.broadcasted_iota(jnp.int32, (8, HD), 0)
        fi = lax.broadcasted_iota(jnp.int32, (8, HD), 1) // C
        sel = (hi == fi).astype(jnp.float32)
        denr = jnp.dot(den8, sel, preferred_element_type=jnp.float32)
        out = jnp.where(denr > 0, nsum / jnp.maximum(denr, 1e-30), 0.0) + b_ref[...]
        mu = jnp.mean(out, axis=-1, keepdims=True)
        var = jnp.mean((out - mu) ** 2, axis=-1, keepdims=True)
        out = (out - mu) * lax.rsqrt(var + 1e-5) * g_ref[...] + beta_ref[...]
        out = jnp.where(out > 0, out, jnp.exp(jnp.minimum(out, 0.0)) - 1.0)
        if proj is None:
            rest[0][...] = out
        else:
            wl_ref, bl_ref, wr_ref, brr_ref, xla_ref, xr_ref = rest
            xla_ref[...] = jnp.dot(out, wl_ref[...], preferred_element_type=jnp.float32) + bl_ref[...]
            xr_ref[...] = jnp.dot(out, wr_ref[...], preferred_element_type=jnp.float32) + brr_ref[...]

    in_specs = [
        pl.BlockSpec((br_rows, HD), lambda i: (i, 0)),
        pl.BlockSpec((br_rows, 8), lambda i: (i, 0)),
        pl.BlockSpec((1, HD), lambda i: (0, 0)),
        pl.BlockSpec((1, HD), lambda i: (0, 0)),
        pl.BlockSpec((1, HD), lambda i: (0, 0)),
    ]
    args = [num, den, b.reshape(1, HD), g.reshape(1, HD), beta.reshape(1, HD)]
    if proj is None:
        out_specs = [pl.BlockSpec((br_rows, HD), lambda i: (i, 0))]
        out_shape = [jax.ShapeDtypeStruct((N, HD), jnp.float32)]
    else:
        Wl, bl, Wr, br = proj
        in_specs += [
            pl.BlockSpec((HD, HD), lambda i: (0, 0)),
            pl.BlockSpec((1, HD), lambda i: (0, 0)),
            pl.BlockSpec((HD, HD), lambda i: (0, 0)),
            pl.BlockSpec((1, HD), lambda i: (0, 0)),
        ]
        args += [Wl, bl.reshape(1, HD), Wr, br.reshape(1, HD)]
        out_specs = [
            pl.BlockSpec((br_rows, HD), lambda i: (i, 0)),
            pl.BlockSpec((br_rows, HD), lambda i: (i, 0)),
        ]
        out_shape = [jax.ShapeDtypeStruct((N, HD), jnp.float32)] * 2

    return pl.pallas_call(
        body,
        grid=(grid,),
        in_specs=in_specs,
        out_specs=out_specs,
        out_shape=out_shape,
    )(*args)


_mesh = plsc.VectorSubcoreMesh(core_axis_name="c", subcore_axis_name="s")


@functools.partial(
    pl.kernel,
    mesh=_mesh,
    out_type=[jax.ShapeDtypeStruct((NC, AR2, HD), jnp.float32)],
    scratch_types=[
        pltpu.VMEM((K,), jnp.int32),
        pltpu.VMEM((K,), jnp.int32),
        pltpu.VMEM((NC * NS, K), jnp.int32),
        pltpu.VMEM((NC * NS, K), jnp.int32),
        pltpu.VMEM((K, HD), jnp.float32),
        pltpu.VMEM((K, HD), jnp.float32),
        pltpu.VMEM((K, HD), jnp.float32),
        pltpu.VMEM((K, HD), jnp.float32),
        pltpu.VMEM((HD,), jnp.float32),
        pltpu.VMEM_SHARED((AR2, HD), jnp.float32),
        pltpu.SemaphoreType.DMA,
        pltpu.SemaphoreType.DMA,
        pltpu.SemaphoreType.DMA,
        pltpu.SemaphoreType.DMA,
        pltpu.SemaphoreType.DMA,
    ],
)
def _sc_edge_pass(src_hbm, dst_hbm, dlocA_hbm, dlocB_hbm, xla_hbm, xr_hbm,
                  ee_hbm, att_hbm, zer_hbm, acc_out,
                  srcv, dstv, dst2vA, dst2vB, xjv, xiv, eev, env, attv, acc_sh,
                  sem1, sem2, sem3, sem4, sem5):
    cid = lax.axis_index("c")
    sid = lax.axis_index("s")
    wid = sid * NC + cid
    rbase = sid * RSUB

    pltpu.sync_copy(zer_hbm.at[pl.ds(0, RSUB)], acc_sh.at[pl.ds(rbase, RSUB)])
    pltpu.sync_copy(zer_hbm.at[pl.ds(0, K)], env)
    pltpu.sync_copy(att_hbm, attv)
    plsc.subcore_barrier()

    att_regs = [attv[pl.ds(j * 16, 16)] for j in range(8)]
    lane = lax.broadcasted_iota(jnp.int32, (16,), 0)
    perms = [lane ^ (1 << p) for p in range(4)]
    ebase = sid * ESUB

    gdn = lax.GatherDimensionNumbers(
        offset_dims=(), collapsed_slice_dims=(0,), start_index_map=(0,))

    def _allsum(v):
        for p in perms:
            v = v + lax.gather(v, p[:, None], dimension_numbers=gdn,
                               slice_sizes=(1,),
                               mode=lax.GatherScatterMode.PROMISE_IN_BOUNDS)
        return v

    def chunk_body(i, carry):
        base = ebase + i * K
        ci1 = pltpu.async_copy(src_hbm.at[pl.ds(base, K)], srcv, sem1)
        ci2 = pltpu.async_copy(dst_hbm.at[pl.ds(base, K)], dstv, sem2)
        ci3 = pltpu.async_copy(dlocA_hbm.at[pl.ds(cid * E + base, K)],
                               dst2vA.at[wid], sem3)
        ci4 = pltpu.async_copy(dlocB_hbm.at[pl.ds(cid * E + base, K)],
                               dst2vB.at[wid], sem4)
        cp3 = pltpu.async_copy(ee_hbm.at[pl.ds(base, K)], eev, sem5)
        ci1.wait()
        cp1 = pltpu.async_copy(xla_hbm.at[srcv], xjv, sem1)
        ci2.wait()
        cp2 = pltpu.async_copy(xr_hbm.at[dstv], xiv, sem2)
        ci3.wait()
        ci4.wait()
        cp1.wait()
        cp2.wait()
        cp3.wait()

        def edge_body(k, carry2):
            xjs = []
            ts = []
            for j in range(8):
                xjj = xjv[k, pl.ds(j * 16, 16)]
                s = xiv[k, pl.ds(j * 16, 16)] + xjj + eev[k, pl.ds(j * 16, 16)]
                s = jnp.maximum(s, s * 0.2)
                xjs.append(xjj)
                ts.append(s * att_regs[j])
            erow = jnp.zeros((16,), jnp.float32)
            for h in range(4):
                a = _allsum(ts[2 * h] + ts[2 * h + 1])
                a = jnp.minimum(jnp.maximum(a, -60.0), 60.0)
                evec = jnp.exp(a)
                xjv[k, pl.ds(2 * h * 16, 16)] = xjs[2 * h] * evec
                xjv[k, pl.ds((2 * h + 1) * 16, 16)] = xjs[2 * h + 1] * evec
                erow = erow + jnp.where(lane == h, evec, 0.0)
            env[k, pl.ds(0, 16)] = erow
            return carry2

        lax.fori_loop(0, K, edge_body, 0)

        cs1 = pltpu.async_copy(xjv, acc_sh.at[dst2vA.at[wid]], sem3, add=True)
        cs2 = pltpu.async_copy(env, acc_sh.at[dst2vB.at[wid]], sem4, add=True)
        cs1.wait()
        cs2.wait()
        return carry

    lax.fori_loop(0, NCHUNK, chunk_body, 0)
    plsc.subcore_barrier()

    pltpu.sync_copy(acc_sh.at[pl.ds(rbase, RSUB)], acc_out.at[cid, pl.ds(rbase, RSUB)])


def kernel(x, edge_index, edge_attr, Wl0, bl0, Wr0, br0, We0, att0, b0, g0, beta0,
           Wl1, bl1, Wr1, br1, We1, att1, b1, g1, beta1):
    src = edge_index[0]
    dst = edge_index[1]
    ea = edge_attr.astype(jnp.float32)

    xla0, xr0 = _proj_pair(x, Wl0, bl0, Wr0, br0)
    ee0, ee1, dlocA, dlocB = _edge_proj(ea, We0, We1, dst)

    zer = jnp.zeros((AR2 // NS, HD), jnp.float32)

    def assemble(acc):
        a3 = acc[:, :2 * HALF].reshape(NC, HALF, 2, HD)
        num = jnp.concatenate([a3[0, :, 0], a3[1, :, 0]], axis=0)[:N]
        den = jnp.concatenate([a3[0, :, 1], a3[1, :, 1]], axis=0)[:N, :8]
        return num, den

    (acc0,) = _sc_edge_pass(src, dst, dlocA, dlocB, xla0, xr0, ee0,
                            att0.reshape(HD), zer)
    num0, den0 = assemble(acc0)
    xla1, xr1 = _post(num0, den0, b0, g0, beta0, proj=(Wl1, bl1, Wr1, br1))
    (acc1,) = _sc_edge_pass(src, dst, dlocA, dlocB, xla1, xr1, ee1,
                            att1.reshape(HD), zer)
    num1, den1 = assemble(acc1)
    (h2,) = _post(num1, den1, b1, g1, beta1, proj=None)
    return h2
